# baseline (device time: 43472 ns/iter reference)
import jax
import jax.numpy as jnp
from jax import lax
from jax.experimental import pallas as pl
from jax.experimental.pallas import tpu as pltpu

N_DEV = 32


def kernel(A, B):
    m, k_per = A.shape
    _, n = B.shape
    rows = m // N_DEV

    def body(a_ref, b_ref, out_ref, partial_ref, gather_ref, result_ref,
             send_sems, recv1_sems, recv2_sems):
        me = lax.axis_index("i")

        partial_ref[...] = jnp.dot(
            a_ref[...], b_ref[...], preferred_element_type=jnp.float32
        ).reshape(N_DEV, rows, n)
        gather_ref[0] = partial_ref[me]

        p1 = []
        for d in range(1, N_DEV):
            tgt = (me + d) % N_DEV
            rdma = pltpu.make_async_remote_copy(
                src_ref=partial_ref.at[tgt],
                dst_ref=gather_ref.at[d],
                send_sem=send_sems.at[d],
                recv_sem=recv1_sems.at[d],
                device_id=(tgt,),
                device_id_type=pl.DeviceIdType.MESH,
            )
            rdma.start()
            p1.append(rdma)
        for rdma in p1:
            rdma.wait_recv()

        acc = jnp.sum(gather_ref[...], axis=0)
        result_ref[me] = jnp.maximum(acc, 0.0)

        p2 = []
        for d in range(1, N_DEV):
            tgt = (me + d) % N_DEV
            p1[d - 1].wait_send()
            rdma = pltpu.make_async_remote_copy(
                src_ref=result_ref.at[me],
                dst_ref=result_ref.at[me],
                send_sem=send_sems.at[d],
                recv_sem=recv2_sems.at[d],
                device_id=(tgt,),
                device_id_type=pl.DeviceIdType.MESH,
            )
            rdma.start()
            p2.append(rdma)
        for d in range(1, N_DEV):
            src = (me - d) % N_DEV
            recv = pltpu.make_async_remote_copy(
                src_ref=result_ref.at[src],
                dst_ref=result_ref.at[src],
                send_sem=send_sems.at[d],
                recv_sem=recv2_sems.at[d],
                device_id=(me,),
                device_id_type=pl.DeviceIdType.MESH,
            )
            recv.wait_recv()

        out_ref[...] = result_ref[...].reshape(m, n)
        for rdma in p2:
            rdma.wait_send()

    return pl.pallas_call(
        body,
        out_shape=jax.ShapeDtypeStruct((m, n), jnp.float32),
        in_specs=[
            pl.BlockSpec(memory_space=pltpu.VMEM),
            pl.BlockSpec(memory_space=pltpu.VMEM),
        ],
        out_specs=pl.BlockSpec(memory_space=pltpu.VMEM),
        scratch_shapes=[
            pltpu.VMEM((N_DEV, rows, n), jnp.float32),
            pltpu.VMEM((N_DEV, rows, n), jnp.float32),
            pltpu.VMEM((N_DEV, rows, n), jnp.float32),
            pltpu.SemaphoreType.DMA((N_DEV,)),
            pltpu.SemaphoreType.DMA((N_DEV,)),
            pltpu.SemaphoreType.DMA((N_DEV,)),
        ],
    )(A, B)


# device time: 32931 ns/iter; 1.3201x vs baseline; 1.3201x over previous
import jax
import jax.numpy as jnp
from jax import lax
from jax.experimental import pallas as pl
from jax.experimental.pallas import tpu as pltpu

N_DEV = 32


def kernel(A, B):
    m, k_per = A.shape
    _, n = B.shape
    rows = m // N_DEV

    def body(a_ref, b_ref, out_ref, partial_ref, gather_ref, result_ref,
             send_sems, recv1_sems, recv2_sems):
        me = lax.axis_index("i")

        partial_ref[...] = jnp.dot(
            a_ref[...], b_ref[...], preferred_element_type=jnp.float32
        ).reshape(N_DEV, rows, n).astype(jnp.bfloat16)
        gather_ref[0] = partial_ref[me]

        p1 = []
        for d in range(1, N_DEV):
            tgt = (me + d) % N_DEV
            rdma = pltpu.make_async_remote_copy(
                src_ref=partial_ref.at[tgt],
                dst_ref=gather_ref.at[d],
                send_sem=send_sems.at[d],
                recv_sem=recv1_sems.at[d],
                device_id=(tgt,),
                device_id_type=pl.DeviceIdType.MESH,
            )
            rdma.start()
            p1.append(rdma)
        for rdma in p1:
            rdma.wait_recv()

        acc = jnp.sum(gather_ref[...].astype(jnp.float32), axis=0)
        result_ref[me] = jnp.maximum(acc, 0.0).astype(jnp.bfloat16)

        p2 = []
        for d in range(1, N_DEV):
            tgt = (me + d) % N_DEV
            p1[d - 1].wait_send()
            rdma = pltpu.make_async_remote_copy(
                src_ref=result_ref.at[me],
                dst_ref=result_ref.at[me],
                send_sem=send_sems.at[d],
                recv_sem=recv2_sems.at[d],
                device_id=(tgt,),
                device_id_type=pl.DeviceIdType.MESH,
            )
            rdma.start()
            p2.append(rdma)
        for d in range(1, N_DEV):
            src = (me - d) % N_DEV
            recv = pltpu.make_async_remote_copy(
                src_ref=result_ref.at[src],
                dst_ref=result_ref.at[src],
                send_sem=send_sems.at[d],
                recv_sem=recv2_sems.at[d],
                device_id=(me,),
                device_id_type=pl.DeviceIdType.MESH,
            )
            recv.wait_recv()

        out_ref[...] = result_ref[...].astype(jnp.float32).reshape(m, n)
        for rdma in p2:
            rdma.wait_send()

    return pl.pallas_call(
        body,
        out_shape=jax.ShapeDtypeStruct((m, n), jnp.float32),
        in_specs=[
            pl.BlockSpec(memory_space=pltpu.VMEM),
            pl.BlockSpec(memory_space=pltpu.VMEM),
        ],
        out_specs=pl.BlockSpec(memory_space=pltpu.VMEM),
        scratch_shapes=[
            pltpu.VMEM((N_DEV, rows, n), jnp.bfloat16),
            pltpu.VMEM((N_DEV, rows, n), jnp.bfloat16),
            pltpu.VMEM((N_DEV, rows, n), jnp.bfloat16),
            pltpu.SemaphoreType.DMA((N_DEV,)),
            pltpu.SemaphoreType.DMA((N_DEV,)),
            pltpu.SemaphoreType.DMA((N_DEV,)),
        ],
    )(A, B)


# device time: 30594 ns/iter; 1.4209x vs baseline; 1.0764x over previous
import jax
import jax.numpy as jnp
from jax import lax
from jax.experimental import pallas as pl
from jax.experimental.pallas import tpu as pltpu

N_DEV = 32
HALVES = 2


def kernel(A, B):
    m, k_per = A.shape
    _, n = B.shape
    rows = m // N_DEV
    half = n // HALVES

    def body(a_ref, b_ref, out_ref, partial_ref, gather_ref, result_ref,
             send_sems, recv1_sems, recv2_sems):
        me = lax.axis_index("i")

        partial_ref[...] = jnp.dot(
            a_ref[...].astype(jnp.bfloat16),
            b_ref[...].astype(jnp.bfloat16),
            preferred_element_type=jnp.float32,
        ).reshape(N_DEV, rows, n).astype(jnp.bfloat16)
        gather_ref[0] = partial_ref[me]

        barrier_sem = pltpu.get_barrier_semaphore()

        @pl.when(me == 0)
        def _():
            pl.semaphore_wait(barrier_sem, N_DEV - 1)
            for j in range(1, N_DEV):
                pl.semaphore_signal(
                    barrier_sem, inc=1,
                    device_id=(j,), device_id_type=pl.DeviceIdType.MESH,
                )

        @pl.when(me != 0)
        def _():
            pl.semaphore_signal(
                barrier_sem, inc=1,
                device_id=(0,), device_id_type=pl.DeviceIdType.MESH,
            )
            pl.semaphore_wait(barrier_sem, 1)

        p1 = {}
        for h in range(HALVES):
            for d in range(1, N_DEV):
                tgt = (me + d) % N_DEV
                rdma = pltpu.make_async_remote_copy(
                    src_ref=partial_ref.at[tgt, :, pl.ds(h * half, half)],
                    dst_ref=gather_ref.at[d, :, pl.ds(h * half, half)],
                    send_sem=send_sems.at[h, d],
                    recv_sem=recv1_sems.at[h, d],
                    device_id=(tgt,),
                    device_id_type=pl.DeviceIdType.MESH,
                )
                rdma.start()
                p1[h, d] = rdma

        p2 = []
        for h in range(HALVES):
            for d in range(1, N_DEV):
                p1[h, d].wait_recv()
            acc = jnp.sum(
                gather_ref[:, :, h * half:(h + 1) * half].astype(jnp.float32),
                axis=0,
            )
            result_ref[me, :, h * half:(h + 1) * half] = jnp.maximum(
                acc, 0.0
            ).astype(jnp.bfloat16)
            for d in range(1, N_DEV):
                tgt = (me + d) % N_DEV
                p1[h, d].wait_send()
                rdma = pltpu.make_async_remote_copy(
                    src_ref=result_ref.at[me, :, pl.ds(h * half, half)],
                    dst_ref=result_ref.at[me, :, pl.ds(h * half, half)],
                    send_sem=send_sems.at[h, d],
                    recv_sem=recv2_sems.at[h, d],
                    device_id=(tgt,),
                    device_id_type=pl.DeviceIdType.MESH,
                )
                rdma.start()
                p2.append(rdma)

        for h in range(HALVES):
            for d in range(1, N_DEV):
                src = (me - d) % N_DEV
                recv = pltpu.make_async_remote_copy(
                    src_ref=result_ref.at[src, :, pl.ds(h * half, half)],
                    dst_ref=result_ref.at[src, :, pl.ds(h * half, half)],
                    send_sem=send_sems.at[h, d],
                    recv_sem=recv2_sems.at[h, d],
                    device_id=(me,),
                    device_id_type=pl.DeviceIdType.MESH,
                )
                recv.wait_recv()

        out_ref[...] = result_ref[...].astype(jnp.float32).reshape(m, n)
        for rdma in p2:
            rdma.wait_send()

    return pl.pallas_call(
        body,
        out_shape=jax.ShapeDtypeStruct((m, n), jnp.float32),
        in_specs=[
            pl.BlockSpec(memory_space=pltpu.VMEM),
            pl.BlockSpec(memory_space=pltpu.VMEM),
        ],
        out_specs=pl.BlockSpec(memory_space=pltpu.VMEM),
        scratch_shapes=[
            pltpu.VMEM((N_DEV, rows, n), jnp.bfloat16),
            pltpu.VMEM((N_DEV, rows, n), jnp.bfloat16),
            pltpu.VMEM((N_DEV, rows, n), jnp.bfloat16),
            pltpu.SemaphoreType.DMA((HALVES, N_DEV)),
            pltpu.SemaphoreType.DMA((HALVES, N_DEV)),
            pltpu.SemaphoreType.DMA((HALVES, N_DEV)),
        ],
        compiler_params=pltpu.CompilerParams(collective_id=0),
    )(A, B)
